# baseline (speedup 1.0000x reference)
import functools

import jax
import jax.numpy as jnp
from jax import lax
from jax.experimental import pallas as pl
from jax.experimental.pallas import tpu as pltpu
from jax.experimental.pallas import tpu_sc as plsc


def _conv_matrix(W):
    Wf = W.reshape(3, 3)
    p = jnp.arange(16)
    h, w = p // 4, p % 4
    dh = h[None, :] - h[:, None]
    dw = w[None, :] - w[:, None]
    valid = (jnp.abs(dh) <= 1) & (jnp.abs(dw) <= 1)
    return jnp.where(valid, Wf[jnp.clip(dh + 1, 0, 2), jnp.clip(dw + 1, 0, 2)], 0.0)


def _body(x_ref, m_ref, o_ref):
    for p in range(16):
        h, w = divmod(p, 4)
        acc = None
        for q in range(16):
            h2, w2 = divmod(q, 4)
            if abs(h2 - h) <= 1 and abs(w2 - w) <= 1:
                t = x_ref[q] * m_ref[p, q]
                acc = t if acc is None else acc + t
        o_ref[p] = jnp.where(x_ref[p] != 0.0, acc, 0.0)


_NC, _NS = 2, 16
_SC_CH = 2048


def _sc_conv(xf, wtaps, n):
    nw = _NC * _NS
    per = n // nw
    ch = _SC_CH
    n_iter = per // ch
    mesh = plsc.VectorSubcoreMesh(
        core_axis_name="c", subcore_axis_name="s", num_cores=_NC,
        num_subcores=_NS)

    @functools.partial(
        pl.kernel,
        mesh=mesh,
        out_type=jax.ShapeDtypeStruct((16 * n,), jnp.float32),
        scratch_types=[
            pltpu.VMEM((16, ch), jnp.float32),
            pltpu.VMEM((16, ch), jnp.float32),
            pltpu.VMEM((9, 16), jnp.float32),
        ],
    )
    def k(x_hbm, w_hbm, o_hbm, xv, ov, wv):
        wid = lax.axis_index("s") * _NC + lax.axis_index("c")
        base0 = wid * per
        pltpu.sync_copy(w_hbm, wv)
        ws = [wv[t] for t in range(9)]

        def chunk(i, carry):
            base = base0 + i * ch
            for p in range(16):
                pltpu.sync_copy(x_hbm.at[pl.ds(p * n + base, ch)], xv.at[p])

            def group(g, carry2):
                off = g * 16
                xs = [xv[q, pl.ds(off, 16)] for q in range(16)]
                for p in range(16):
                    h, w = divmod(p, 4)
                    acc = None
                    for dh in (-1, 0, 1):
                        for dw in (-1, 0, 1):
                            h2, w2 = h + dh, w + dw
                            if 0 <= h2 < 4 and 0 <= w2 < 4:
                                t = ws[(dh + 1) * 3 + dw + 1] * xs[h2 * 4 + w2]
                                acc = t if acc is None else acc + t
                    zero = jnp.zeros((16,), jnp.float32)
                    ov[p, pl.ds(off, 16)] = jnp.where(xs[p] != 0.0, acc, zero)
                return carry2

            lax.fori_loop(0, ch // 16, group, 0)
            for p in range(16):
                pltpu.sync_copy(ov.at[p], o_hbm.at[pl.ds(p * n + base, ch)])
            return carry

        lax.fori_loop(0, n_iter, chunk, 0)

    return k(xf, wtaps)


def kernel(x, W):
    N = x.shape[0]
    xflat = x.transpose(1, 2, 3, 0).reshape(16 * N)
    wtaps = jnp.broadcast_to(W.reshape(9, 1), (9, 16))
    oflat = _sc_conv(xflat, wtaps, N)
    return oflat.reshape(4, 4, 1, N).transpose(3, 0, 1, 2)


def _tc_kernel(x, W):
    N = x.shape[0]
    xt = x.transpose(1, 2, 3, 0).reshape(16, N // 128, 128)
    M = _conv_matrix(W)

    BM = 1024
    rows = N // 128
    out = pl.pallas_call(
        _body,
        grid=(rows // BM,),
        in_specs=[
            pl.BlockSpec((16, BM, 128), lambda i: (0, i, 0)),
            pl.BlockSpec(memory_space=pltpu.SMEM),
        ],
        out_specs=pl.BlockSpec((16, BM, 128), lambda i: (0, i, 0)),
        out_shape=jax.ShapeDtypeStruct((16, rows, 128), jnp.float32),
    )(xt, M)
    return out.reshape(4, 4, 1, N).transpose(3, 0, 1, 2)


# speedup vs baseline: 6.8183x; 6.8183x over previous
import functools

import jax
import jax.numpy as jnp
from jax import lax
from jax.experimental import pallas as pl
from jax.experimental.pallas import tpu as pltpu
from jax.experimental.pallas import tpu_sc as plsc


def _conv_matrix(W):
    Wf = W.reshape(3, 3)
    p = jnp.arange(16)
    h, w = p // 4, p % 4
    dh = h[None, :] - h[:, None]
    dw = w[None, :] - w[:, None]
    valid = (jnp.abs(dh) <= 1) & (jnp.abs(dw) <= 1)
    return jnp.where(valid, Wf[jnp.clip(dh + 1, 0, 2), jnp.clip(dw + 1, 0, 2)], 0.0)


def _body(x_ref, m_ref, o_ref):
    xb = [x_ref[q].astype(jnp.bfloat16) for q in range(16)]
    for p in range(16):
        h, w = divmod(p, 4)
        acc = None
        for q in range(16):
            h2, w2 = divmod(q, 4)
            if abs(h2 - h) <= 1 and abs(w2 - w) <= 1:
                t = xb[q] * m_ref[p, q].astype(jnp.bfloat16)
                acc = t if acc is None else acc + t
        o_ref[p] = jnp.where(x_ref[p] != 0.0, acc.astype(jnp.float32), 0.0)


_NC, _NS = 2, 16
_SC_CH = 2048


def _sc_conv(xf, wtaps, n):
    nw = _NC * _NS
    per = n // nw
    ch = _SC_CH
    n_iter = per // ch
    mesh = plsc.VectorSubcoreMesh(
        core_axis_name="c", subcore_axis_name="s", num_cores=_NC,
        num_subcores=_NS)

    @functools.partial(
        pl.kernel,
        mesh=mesh,
        out_type=jax.ShapeDtypeStruct((16 * n,), jnp.float32),
        scratch_types=[
            pltpu.VMEM((16, ch), jnp.float32),
            pltpu.VMEM((16, ch), jnp.float32),
            pltpu.VMEM((9, 16), jnp.float32),
        ],
    )
    def k(x_hbm, w_hbm, o_hbm, xv, ov, wv):
        wid = lax.axis_index("s") * _NC + lax.axis_index("c")
        base0 = wid * per
        pltpu.sync_copy(w_hbm, wv)
        ws = [wv[t] for t in range(9)]

        def chunk(i, carry):
            base = base0 + i * ch
            for p in range(16):
                pltpu.sync_copy(x_hbm.at[pl.ds(p * n + base, ch)], xv.at[p])

            def group(g, carry2):
                off = g * 16
                xs = [xv[q, pl.ds(off, 16)] for q in range(16)]
                for p in range(16):
                    h, w = divmod(p, 4)
                    acc = None
                    for dh in (-1, 0, 1):
                        for dw in (-1, 0, 1):
                            h2, w2 = h + dh, w + dw
                            if 0 <= h2 < 4 and 0 <= w2 < 4:
                                t = ws[(dh + 1) * 3 + dw + 1] * xs[h2 * 4 + w2]
                                acc = t if acc is None else acc + t
                    zero = jnp.zeros((16,), jnp.float32)
                    ov[p, pl.ds(off, 16)] = jnp.where(xs[p] != 0.0, acc, zero)
                return carry2

            lax.fori_loop(0, ch // 16, group, 0)
            for p in range(16):
                pltpu.sync_copy(ov.at[p], o_hbm.at[pl.ds(p * n + base, ch)])
            return carry

        lax.fori_loop(0, n_iter, chunk, 0)

    return k(xf, wtaps)


def _sc_kernel(x, W):
    N = x.shape[0]
    xflat = x.transpose(1, 2, 3, 0).reshape(16 * N)
    wtaps = jnp.broadcast_to(W.reshape(9, 1), (9, 16))
    oflat = _sc_conv(xflat, wtaps, N)
    return oflat.reshape(4, 4, 1, N).transpose(3, 0, 1, 2)


def kernel(x, W):
    N = x.shape[0]
    xt = x.transpose(1, 2, 3, 0).reshape(16, N // 128, 128)
    M = _conv_matrix(W)

    BM = 1024
    rows = N // 128
    out = pl.pallas_call(
        _body,
        grid=(rows // BM,),
        in_specs=[
            pl.BlockSpec((16, BM, 128), lambda i: (0, i, 0)),
            pl.BlockSpec(memory_space=pltpu.SMEM),
        ],
        out_specs=pl.BlockSpec((16, BM, 128), lambda i: (0, i, 0)),
        out_shape=jax.ShapeDtypeStruct((16, rows, 128), jnp.float32),
    )(xt, M)
    return out.reshape(4, 4, 1, N).transpose(3, 0, 1, 2)
